# baseline (speedup 1.0000x reference)
import functools

import jax
import jax.numpy as jnp
from jax.experimental import pallas as pl
from jax.experimental.pallas import tpu as pltpu

_SPLIT = 2


def _gap_body(x_ref, o_ref, *, inv):
    j = pl.program_id(1)
    s = jnp.sum(x_ref[...], axis=0, keepdims=True)[None]

    @pl.when(j == 0)
    def _():
        o_ref[...] = s

    @pl.when((j > 0) & (j < _SPLIT - 1))
    def _():
        o_ref[...] += s

    @pl.when(j == _SPLIT - 1)
    def _():
        o_ref[...] = (o_ref[...] + s) * inv


def kernel(x):
    b, c, h, w = x.shape
    hw = h * w
    rows_per_block = hw // _SPLIT
    x2 = jnp.transpose(x, (0, 2, 3, 1)).reshape(b * hw, c)
    out = pl.pallas_call(
        functools.partial(_gap_body, inv=1.0 / hw),
        out_shape=jax.ShapeDtypeStruct((b, 1, c), x.dtype),
        grid=(b, _SPLIT),
        in_specs=[pl.BlockSpec((rows_per_block, c), lambda i, j: (i * _SPLIT + j, 0))],
        out_specs=pl.BlockSpec((1, 1, c), lambda i, j: (i, 0, 0)),
        compiler_params=pltpu.CompilerParams(
            dimension_semantics=("arbitrary", "arbitrary"),
            vmem_limit_bytes=50 * 1024 * 1024,
        ),
    )(x2)
    return out.reshape(b, c)


# speedup vs baseline: 1.0090x; 1.0090x over previous
import functools

import jax
import jax.numpy as jnp
from jax.experimental import pallas as pl
from jax.experimental.pallas import tpu as pltpu

_BPB = 2


def _gap_body(x_ref, o_ref, *, hw, inv):
    x = x_ref[...].reshape(_BPB, hw, x_ref.shape[1])
    o_ref[...] = jnp.sum(x, axis=1, keepdims=True) * inv


def kernel(x):
    b, c, h, w = x.shape
    hw = h * w
    x2 = jnp.transpose(x, (0, 2, 3, 1)).reshape(b * hw, c)
    out = pl.pallas_call(
        functools.partial(_gap_body, hw=hw, inv=1.0 / hw),
        out_shape=jax.ShapeDtypeStruct((b, 1, c), x.dtype),
        grid=(b // _BPB,),
        in_specs=[pl.BlockSpec((_BPB * hw, c), lambda i: (i, 0))],
        out_specs=pl.BlockSpec((_BPB, 1, c), lambda i: (i, 0, 0)),
        compiler_params=pltpu.CompilerParams(
            dimension_semantics=("arbitrary",),
            vmem_limit_bytes=56 * 1024 * 1024,
        ),
    )(x2)
    return out.reshape(b, c)


# speedup vs baseline: 1.0159x; 1.0068x over previous
import jax
import jax.numpy as jnp
from jax.experimental import pallas as pl
from jax.experimental.pallas import tpu as pltpu


def _gap_body(x_ref, o_ref):
    inv = 1.0 / x_ref.shape[0]
    o_ref[0, ...] = jnp.sum(x_ref[...], axis=0, keepdims=True) * inv


def kernel(x):
    b, c, h, w = x.shape
    hw = h * w
    x2 = jnp.transpose(x, (0, 2, 3, 1)).reshape(b * hw, c)
    out = pl.pallas_call(
        _gap_body,
        out_shape=jax.ShapeDtypeStruct((b, 1, c), x.dtype),
        grid=(b,),
        in_specs=[pl.BlockSpec((hw, c), lambda i: (i, 0))],
        out_specs=pl.BlockSpec((1, 1, c), lambda i: (i, 0, 0)),
        compiler_params=pltpu.CompilerParams(
            dimension_semantics=("arbitrary",),
            vmem_limit_bytes=50 * 1024 * 1024,
        ),
    )(x2)
    return out.reshape(b, c)


# speedup vs baseline: 1.0168x; 1.0010x over previous
import jax
import jax.numpy as jnp
from jax.experimental import pallas as pl
from jax.experimental.pallas import tpu as pltpu


def _gap_body(xa_ref, xb_ref, o_ref):
    inv = 1.0 / (2 * xa_ref.shape[0])
    s = jnp.sum(xa_ref[...], axis=0, keepdims=True) + jnp.sum(
        xb_ref[...], axis=0, keepdims=True
    )
    o_ref[0, ...] = s * inv


def kernel(x):
    b, c, h, w = x.shape
    hw = h * w
    half = hw // 2
    x2 = jnp.transpose(x, (0, 2, 3, 1)).reshape(b * hw, c)
    out = pl.pallas_call(
        _gap_body,
        out_shape=jax.ShapeDtypeStruct((b, 1, c), x.dtype),
        grid=(b,),
        in_specs=[
            pl.BlockSpec((half, c), lambda i: (2 * i, 0)),
            pl.BlockSpec((half, c), lambda i: (2 * i + 1, 0)),
        ],
        out_specs=pl.BlockSpec((1, 1, c), lambda i: (i, 0, 0)),
        compiler_params=pltpu.CompilerParams(
            dimension_semantics=("arbitrary",),
            vmem_limit_bytes=50 * 1024 * 1024,
        ),
    )(x2, x2)
    return out.reshape(b, c)
